# Initial kernel scaffold; baseline (speedup 1.0000x reference)
import jax
import jax.numpy as jnp
from jax.experimental import pallas as pl


def kernel(x, pos_table):
    raise NotImplementedError("write your pallas kernel here")


# speedup vs baseline: 1.6818x; 1.6818x over previous
import jax
import jax.numpy as jnp
from jax.experimental import pallas as pl
from jax.experimental.pallas import tpu as pltpu

BLOCK_S = 512


def _add_kernel(x_ref, pos_ref, out_ref):
    out_ref[0] = x_ref[0] + pos_ref[...]


def kernel(x, pos_table):
    batch, seq_len, d = x.shape
    grid = (seq_len // BLOCK_S, batch)
    return pl.pallas_call(
        _add_kernel,
        grid=grid,
        in_specs=[
            pl.BlockSpec((1, BLOCK_S, d), lambda s, b: (b, s, 0)),
            pl.BlockSpec((BLOCK_S, d), lambda s, b: (s, 0)),
        ],
        out_specs=pl.BlockSpec((1, BLOCK_S, d), lambda s, b: (b, s, 0)),
        out_shape=jax.ShapeDtypeStruct((batch, seq_len, d), x.dtype),
        compiler_params=pltpu.CompilerParams(
            dimension_semantics=("arbitrary", "arbitrary"),
        ),
    )(x, pos_table)


# speedup vs baseline: 1.8748x; 1.1148x over previous
import jax
import jax.numpy as jnp
from jax.experimental import pallas as pl
from jax.experimental.pallas import tpu as pltpu

BLOCK_S = 1024


def _add_kernel(x_ref, pos_ref, out_ref):
    out_ref[0] = x_ref[0] + pos_ref[...]


def kernel(x, pos_table):
    batch, seq_len, d = x.shape
    grid = (seq_len // BLOCK_S, batch)
    return pl.pallas_call(
        _add_kernel,
        grid=grid,
        in_specs=[
            pl.BlockSpec((1, BLOCK_S, d), lambda s, b: (b, s, 0)),
            pl.BlockSpec((BLOCK_S, d), lambda s, b: (s, 0)),
        ],
        out_specs=pl.BlockSpec((1, BLOCK_S, d), lambda s, b: (b, s, 0)),
        out_shape=jax.ShapeDtypeStruct((batch, seq_len, d), x.dtype),
        compiler_params=pltpu.CompilerParams(
            dimension_semantics=("arbitrary", "arbitrary"),
        ),
    )(x, pos_table)


# speedup vs baseline: 1.9909x; 1.0619x over previous
import jax
import jax.numpy as jnp
from jax.experimental import pallas as pl
from jax.experimental.pallas import tpu as pltpu

BLOCK_S = 2048


def _add_kernel(x_ref, pos_ref, out_ref):
    out_ref[0] = x_ref[0] + pos_ref[...]


def kernel(x, pos_table):
    batch, seq_len, d = x.shape
    grid = (seq_len // BLOCK_S, batch)
    return pl.pallas_call(
        _add_kernel,
        grid=grid,
        in_specs=[
            pl.BlockSpec((1, BLOCK_S, d), lambda s, b: (b, s, 0)),
            pl.BlockSpec((BLOCK_S, d), lambda s, b: (s, 0)),
        ],
        out_specs=pl.BlockSpec((1, BLOCK_S, d), lambda s, b: (b, s, 0)),
        out_shape=jax.ShapeDtypeStruct((batch, seq_len, d), x.dtype),
        compiler_params=pltpu.CompilerParams(
            dimension_semantics=("arbitrary", "arbitrary"),
        ),
    )(x, pos_table)


# speedup vs baseline: 1.9933x; 1.0012x over previous
import jax
import jax.numpy as jnp
from jax.experimental import pallas as pl
from jax.experimental.pallas import tpu as pltpu

BLOCK_S = 2048


def _add_kernel(x_ref, pos_ref, out_ref):
    out_ref[0] = x_ref[0] + pos_ref[...]


def kernel(x, pos_table):
    batch, seq_len, d = x.shape
    grid = (seq_len // BLOCK_S, batch)
    return pl.pallas_call(
        _add_kernel,
        grid=grid,
        in_specs=[
            pl.BlockSpec((1, BLOCK_S, d), lambda s, b: (b, s, 0)),
            pl.BlockSpec((BLOCK_S, d), lambda s, b: (s, 0)),
        ],
        out_specs=pl.BlockSpec((1, BLOCK_S, d), lambda s, b: (b, s, 0)),
        out_shape=jax.ShapeDtypeStruct((batch, seq_len, d), x.dtype),
        compiler_params=pltpu.CompilerParams(
            dimension_semantics=("parallel", "parallel"),
        ),
    )(x, pos_table)
